# baseline (speedup 1.0000x reference)
import functools

import jax
import jax.numpy as jnp
from jax import lax
from jax.experimental import pallas as pl
from jax.experimental.pallas import tpu as pltpu
from jax.experimental.pallas import tpu_sc as plsc

_BATCH = 16384
_DIM = 64
_NODES = 1000000
_HALF_PAIRS = _NODES // 4

_info = plsc.get_sparse_core_info()
_NC = _info.num_cores
_NW = _NC * _info.num_subcores
_B_PER_W = _BATCH // _NW
_CHUNK = 128
_ROUND = 256

_mesh = plsc.VectorSubcoreMesh(core_axis_name="c", subcore_axis_name="s")


@functools.partial(
    pl.kernel,
    mesh=_mesh,
    out_type=jax.ShapeDtypeStruct((_BATCH, 4 * _DIM), jnp.float32),
    scratch_types=[
        pltpu.VMEM((_B_PER_W,), jnp.int32),
        pltpu.VMEM((_B_PER_W,), jnp.int32),
        pltpu.VMEM((_ROUND, 4 * _DIM), jnp.float32),
        pltpu.SemaphoreType.DMA,
    ],
)
def _sc_gather(pa_hbm, pb_hbm, tabA_hbm, tabB_hbm, inter_hbm,
               pa_v, pb_v, rows_v, sem):
    wid = lax.axis_index("s") * _NC + lax.axis_index("c")
    base = wid * _B_PER_W
    pltpu.sync_copy(pa_hbm.at[pl.ds(base, _B_PER_W)], pa_v)
    pltpu.sync_copy(pb_hbm.at[pl.ds(base, _B_PER_W)], pb_v)
    for rnd in range(_B_PER_W // _ROUND):
        r0 = rnd * _ROUND
        copies = []
        for m in range(_ROUND // _CHUNK):
            o = r0 + m * _CHUNK
            copies.append(pltpu.async_copy(
                tabA_hbm.at[pa_v.at[pl.ds(o, _CHUNK)]],
                rows_v.at[pl.ds(m * _CHUNK, _CHUNK), pl.ds(0, 2 * _DIM)],
                sem))
            copies.append(pltpu.async_copy(
                tabB_hbm.at[pb_v.at[pl.ds(o, _CHUNK)]],
                rows_v.at[pl.ds(m * _CHUNK, _CHUNK), pl.ds(2 * _DIM, 2 * _DIM)],
                sem))
        for c in copies:
            c.wait()
        pltpu.sync_copy(rows_v, inter_hbm.at[pl.ds(base + r0, _ROUND)])


_TC_BLK = 256


def _tc_select_body(sel_ref, rows_ref, out_ref):
    s = jnp.reshape(sel_ref[0, 0, :], (_TC_BLK, 1))
    a = jnp.where(s % 2 == 1, rows_ref[:, _DIM:2 * _DIM], rows_ref[:, :_DIM])
    b = jnp.where(s % 2 == 1, rows_ref[:, 3 * _DIM:], rows_ref[:, 2 * _DIM:3 * _DIM])
    out_ref[...] = jnp.where(s >= 2, b, a)


_tc_select = pl.pallas_call(
    _tc_select_body,
    grid=(_BATCH // _TC_BLK,),
    in_specs=[
        pl.BlockSpec((1, 1, _TC_BLK), lambda i: (i, 0, 0)),
        pl.BlockSpec((_TC_BLK, 4 * _DIM), lambda i: (i, 0)),
    ],
    out_specs=pl.BlockSpec((_TC_BLK, _DIM), lambda i: (i, 0)),
    out_shape=jax.ShapeDtypeStruct((_BATCH, _DIM), jnp.float32),
)


def kernel(n_id, emb_table):
    n_id = n_id.astype(jnp.int32)
    tabA = emb_table[: _NODES // 2].reshape(_HALF_PAIRS, 2 * _DIM)
    tabB = emb_table[_NODES // 2 :].reshape(_HALF_PAIRS, 2 * _DIM)
    pid = lax.shift_right_logical(n_id, 1)
    pa = jnp.minimum(pid, _HALF_PAIRS - 1)
    pb = jnp.clip(pid - _HALF_PAIRS, 0, _HALF_PAIRS - 1)
    rows = _sc_gather(pa, pb, tabA, tabB)
    sel = lax.bitwise_and(n_id, 1) + 2 * (pid >= _HALF_PAIRS).astype(jnp.int32)
    sel3 = sel.reshape(_BATCH // _TC_BLK, 1, _TC_BLK)
    return _tc_select(sel3, rows)


# speedup vs baseline: 2.1644x; 2.1644x over previous
import functools

import jax
import jax.numpy as jnp
from jax import lax
from jax.experimental import pallas as pl
from jax.experimental.pallas import tpu as pltpu
from jax.experimental.pallas import tpu_sc as plsc

_BATCH = 16384
_DIM = 64
_NODES = 1000000

_info = plsc.get_sparse_core_info()
_NC = _info.num_cores
_NW = _NC * _info.num_subcores
_B_PER_W = _BATCH // _NW
_CHUNK = 128

_mesh = plsc.VectorSubcoreMesh(core_axis_name="c", subcore_axis_name="s")


@functools.partial(
    pl.kernel,
    mesh=_mesh,
    out_type=jax.ShapeDtypeStruct((_BATCH, 2 * _DIM), jnp.float32),
    scratch_types=[
        pltpu.VMEM((_B_PER_W,), jnp.int32),
        pltpu.VMEM((_B_PER_W, 2 * _DIM), jnp.float32),
        pltpu.SemaphoreType.DMA,
    ],
)
def _sc_gather(idx_hbm, padded_hbm, out_hbm, idx_v, rows_v, sem):
    wid = lax.axis_index("s") * _NC + lax.axis_index("c")
    base = wid * _B_PER_W
    pltpu.sync_copy(idx_hbm.at[pl.ds(base, _B_PER_W)], idx_v)
    copies = []
    for m in range(_B_PER_W // _CHUNK):
        src = padded_hbm.at[idx_v.at[pl.ds(m * _CHUNK, _CHUNK)]]
        dst = rows_v.at[pl.ds(m * _CHUNK, _CHUNK), :]
        copies.append(pltpu.async_copy(src, dst, sem))
    for c in copies:
        c.wait()
    pltpu.sync_copy(rows_v, out_hbm.at[pl.ds(base, _B_PER_W)])


_TC_BLK = 256


def _tc_slice_body(rows_ref, out_ref):
    out_ref[...] = rows_ref[:, : _DIM]


_tc_slice = pl.pallas_call(
    _tc_slice_body,
    grid=(_BATCH // _TC_BLK,),
    in_specs=[pl.BlockSpec((_TC_BLK, 2 * _DIM), lambda i: (i, 0))],
    out_specs=pl.BlockSpec((_TC_BLK, _DIM), lambda i: (i, 0)),
    out_shape=jax.ShapeDtypeStruct((_BATCH, _DIM), jnp.float32),
)


def kernel(n_id, emb_table):
    padded = jnp.pad(emb_table, ((0, 0), (0, _DIM)))
    rows = _sc_gather(n_id.astype(jnp.int32), padded)
    return _tc_slice(rows)


# speedup vs baseline: 2.2979x; 1.0617x over previous
import functools

import jax
import jax.numpy as jnp
from jax import lax
from jax.experimental import pallas as pl
from jax.experimental.pallas import tpu as pltpu
from jax.experimental.pallas import tpu_sc as plsc

_BATCH = 16384
_DIM = 64
_NODES = 1000000

_info = plsc.get_sparse_core_info()
_NC = _info.num_cores
_NW = _NC * _info.num_subcores
_B_PER_W = _BATCH // _NW
_CHUNK = 128

_mesh = plsc.VectorSubcoreMesh(core_axis_name="c", subcore_axis_name="s")


@functools.partial(
    pl.kernel,
    mesh=_mesh,
    out_type=jax.ShapeDtypeStruct((_BATCH, 2 * _DIM), jnp.float32),
    scratch_types=[
        pltpu.VMEM((_B_PER_W,), jnp.int32),
        pltpu.VMEM((_B_PER_W, 2 * _DIM), jnp.float32),
        pltpu.SemaphoreType.DMA,
    ],
)
def _sc_gather(idx_hbm, padded_hbm, out_hbm, idx_v, rows_v, sem):
    wid = lax.axis_index("s") * _NC + lax.axis_index("c")
    base = wid * _B_PER_W
    pltpu.sync_copy(idx_hbm.at[pl.ds(base, _B_PER_W)], idx_v)
    copies = []
    for m in range(_B_PER_W // _CHUNK):
        src = padded_hbm.at[idx_v.at[pl.ds(m * _CHUNK, _CHUNK)]]
        dst = rows_v.at[pl.ds(m * _CHUNK, _CHUNK), :]
        copies.append(pltpu.async_copy(src, dst, sem))
    for c in copies:
        c.wait()
    pltpu.sync_copy(rows_v, out_hbm.at[pl.ds(base, _B_PER_W)])


def kernel(n_id, emb_table):
    padded = jnp.pad(emb_table, ((0, 0), (0, _DIM)))
    rows = _sc_gather(n_id.astype(jnp.int32), padded)
    return rows[:, : _DIM]
